# Initial kernel scaffold; baseline (speedup 1.0000x reference)
import jax
import jax.numpy as jnp
from jax.experimental import pallas as pl


def kernel(x, h, edges, edge_weights, time_embed, message_params, coord_params, inv_params, Wa, ba, Wb, bb):
    raise NotImplementedError("write your pallas kernel here")


# speedup vs baseline: 25.5905x; 25.5905x over previous
import functools

import jax
import jax.numpy as jnp
from jax.experimental import pallas as pl
from jax.experimental.pallas import tpu as pltpu

_B, _N, _E = 32, 128, 16256
_EC = 2032
_NC = _E // _EC
_EPS = 1e-3
_SQRT2 = 1.4142135623730951


def _gelu(v):
    return 0.5 * v * (1.0 + jax.lax.erf(v / _SQRT2))


def _fold_bn(p):
    gamma, beta, mm, mv, w, b = p
    s = gamma / jnp.sqrt(mv + _EPS)
    t = beta - mm * s
    return s[:, None] * w, t @ w + b


def _body(hx_ref, nidx_ref, bidx_ref, seg_ref, tb_ref, w20_ref, wp_ref,
          cr_ref, ox_ref, oh_ref, tn_s, tb_s, acc_s):
    c = pl.program_id(1)
    nc = pl.num_programs(1)

    @pl.when(c == 0)
    def _init():
        hxb = hx_ref[0]
        bias1 = tb_ref[0, 0, 0:16][None, :]
        pn = jnp.dot(hxb, w20_ref[:, 0:16],
                     preferred_element_type=jnp.float32) + bias1
        pb = jnp.dot(hxb, w20_ref[:, 16:32],
                     preferred_element_type=jnp.float32)
        zeros4 = jnp.zeros((_N, 4), jnp.float32)
        tn_s[...] = jnp.concatenate([pn, hxb[:, 16:20], zeros4], axis=1)
        tb_s[...] = jnp.concatenate([pb, hxb[:, 16:20], zeros4], axis=1)
        acc_s[...] = jnp.zeros((_N, 24), jnp.float32)

    ids_n = nidx_ref[0, 0, 0, :]
    ids_b = bidx_ref[0, 0, 0, :]
    seg = seg_ref[0, :]

    lane = jax.lax.broadcasted_iota(jnp.int32, (_EC, _N), 1)
    oh_n = (ids_n[:, None] == lane).astype(jnp.float32)
    oh_b = (ids_b[:, None] == lane).astype(jnp.float32)

    gn = jnp.dot(oh_n, tn_s[...], preferred_element_type=jnp.float32, precision=jax.lax.Precision.HIGHEST)
    gb = jnp.dot(oh_b, tb_s[...], preferred_element_type=jnp.float32, precision=jax.lax.Precision.HIGHEST)

    z1 = _gelu(gn[:, 0:16] + gb[:, 0:16])
    msg = _gelu(jnp.dot(z1, wp_ref[:, 0:16],
                        preferred_element_type=jnp.float32)
                + cr_ref[0, 0:16][None, :])
    cfz = _gelu(jnp.dot(msg, wp_ref[:, 16:32],
                        preferred_element_type=jnp.float32)
                + tb_ref[0, 0, 16:32][None, :])
    cf = _gelu(jnp.dot(cfz, wp_ref[:, 82:83],
                       preferred_element_type=jnp.float32) + cr_ref[0, 32])
    ab = _gelu(jnp.dot(msg, wp_ref[:, 80:82],
                       preferred_element_type=jnp.float32)
               + tb_ref[0, 0, 48:50][None, :])
    cu = cf * (ab[:, 0:1] * gn[:, 16:20] + ab[:, 1:2] * gb[:, 16:20])

    sub = jax.lax.broadcasted_iota(jnp.int32, (_N, _EC), 0)
    oh_s = (sub == seg).astype(jnp.float32)
    scat = jnp.concatenate(
        [msg, cu, jnp.ones((_EC, 1), jnp.float32),
         jnp.zeros((_EC, 3), jnp.float32)], axis=1)
    acc_s[...] += jnp.dot(oh_s, scat, preferred_element_type=jnp.float32, precision=jax.lax.Precision.HIGHEST)

    @pl.when(c == nc - 1)
    def _fin():
        aggm = acc_s[:, 0:16]
        aggc = acc_s[:, 16:20]
        cnt = acc_s[:, 20:21]
        hxb = hx_ref[0]
        ox_ref[0] = hxb[:, 16:20] + jnp.where(
            cnt > 0.0, aggc / jnp.maximum(cnt, 1.0), 0.0)
        zi = _gelu(jnp.dot(hxb[:, 0:16], wp_ref[:, 48:64],
                           preferred_element_type=jnp.float32)
                   + jnp.dot(aggm, wp_ref[:, 64:80],
                             preferred_element_type=jnp.float32)
                   + tb_ref[0, 0, 32:48][None, :])
        oh_ref[0] = _gelu(jnp.dot(zi, wp_ref[:, 32:48],
                                  preferred_element_type=jnp.float32)
                          + cr_ref[0, 16:32][None, :])


@functools.partial(jax.jit, static_argnames=())
def kernel(x, h, edges, edge_weights, time_embed, message_params,
           coord_params, inv_params, Wa, ba, Wb, bb):
    del edge_weights
    w1p, b1p = _fold_bn(message_params[0])
    w2p, b2p = _fold_bn(message_params[1])
    wc1p, bc1p = _fold_bn(coord_params[0])
    wc2p, bc2p = _fold_bn(coord_params[1])
    wi1p, bi1p = _fold_bn(inv_params[0])
    wi2p, bi2p = _fold_bn(inv_params[1])

    bias1_t = time_embed @ w1p[40:48] + b1p
    biasc1_t = time_embed @ wc1p[16:24] + bc1p
    biasi_t = time_embed @ wi1p[32:40] + bi1p
    ca = time_embed @ Wa[16:24] + ba
    cb = time_embed @ Wb[16:24] + bb
    tb = jnp.concatenate(
        [bias1_t, biasc1_t, biasi_t, ca, cb,
         jnp.zeros((_B, 14), jnp.float32)], axis=1).reshape(_B, 1, 64)

    w20 = jnp.concatenate(
        [jnp.concatenate([w1p[0:16], w1p[32:36]], axis=0),
         jnp.concatenate([w1p[16:32], w1p[36:40]], axis=0)], axis=1)
    wp = jnp.concatenate(
        [w2p, wc1p[0:16], wi2p, wi1p[0:16], wi1p[16:32],
         Wa[0:16], Wb[0:16], wc2p,
         jnp.zeros((16, 13), jnp.float32)], axis=1)
    cr = jnp.concatenate(
        [b2p, bi2p, bc2p, jnp.zeros((31,), jnp.float32)])[None, :]

    hx = jnp.concatenate([h, x], axis=2)
    nidx = edges[:, :, 0].reshape(_B, _NC, 1, _EC)
    bidx = edges[:, :, 1].reshape(_B, _NC, 1, _EC)
    seg = edges[0, :, 0].reshape(_NC, 1, _EC)

    grid = (_B, _NC)
    ox, oh = pl.pallas_call(
        _body,
        grid=grid,
        in_specs=[
            pl.BlockSpec((1, _N, 20), lambda b, c: (b, 0, 0)),
            pl.BlockSpec((1, 1, 1, _EC), lambda b, c: (b, c, 0, 0)),
            pl.BlockSpec((1, 1, 1, _EC), lambda b, c: (b, c, 0, 0)),
            pl.BlockSpec((1, 1, _EC), lambda b, c: (c, 0, 0)),
            pl.BlockSpec((1, 1, 64), lambda b, c: (b, 0, 0)),
            pl.BlockSpec((20, 32), lambda b, c: (0, 0)),
            pl.BlockSpec((16, 96), lambda b, c: (0, 0)),
            pl.BlockSpec((1, 64), lambda b, c: (0, 0)),
        ],
        out_specs=[
            pl.BlockSpec((1, _N, 4), lambda b, c: (b, 0, 0)),
            pl.BlockSpec((1, _N, 16), lambda b, c: (b, 0, 0)),
        ],
        out_shape=[
            jax.ShapeDtypeStruct((_B, _N, 4), jnp.float32),
            jax.ShapeDtypeStruct((_B, _N, 16), jnp.float32),
        ],
        scratch_shapes=[
            pltpu.VMEM((_N, 24), jnp.float32),
            pltpu.VMEM((_N, 24), jnp.float32),
            pltpu.VMEM((_N, 24), jnp.float32),
        ],
        compiler_params=pltpu.CompilerParams(
            dimension_semantics=("arbitrary", "arbitrary")),
    )(hx, nidx, bidx, seg, tb, w20, wp, cr)
    return (ox, oh)


# speedup vs baseline: 29.3692x; 1.1477x over previous
import jax
import jax.numpy as jnp
from jax.experimental import pallas as pl
from jax.experimental.pallas import tpu as pltpu

_B, _N, _E = 32, 128, 16256
_EC = 4064
_NC = _E // _EC
_EPS = 1e-3
_SQRT2 = 1.4142135623730951


def _gelu(v):
    return 0.5 * v * (1.0 + jax.lax.erf(v / _SQRT2))


def _fold_bn(p):
    gamma, beta, mm, mv, w, b = p
    s = gamma / jnp.sqrt(mv + _EPS)
    t = beta - mm * s
    return s[:, None] * w, t @ w + b


def _body(hx_ref, nidx_ref, bidx_ref, seg_ref, tb_ref, w20_ref, wp_ref,
          wcat_ref, cr_ref, ox_ref, oh_ref, thi_s, tlo_s, acc_s):
    c = pl.program_id(1)
    nc = pl.num_programs(1)

    @pl.when(c == 0)
    def _init():
        hxb = hx_ref[0]
        bias1 = tb_ref[0, 0, 0:16][None, :]
        pn = jnp.dot(hxb, w20_ref[:, 0:16],
                     preferred_element_type=jnp.float32) + bias1
        pb = jnp.dot(hxb, w20_ref[:, 16:32],
                     preferred_element_type=jnp.float32)
        zeros4 = jnp.zeros((_N, 4), jnp.float32)
        xb = hxb[:, 16:20]
        tn = jnp.concatenate([pn, xb, zeros4], axis=1)
        tbl = jnp.concatenate([pb, zeros4, xb], axis=1)
        tfull = jnp.concatenate([tn, tbl], axis=0)
        hi = tfull.astype(jnp.bfloat16)
        thi_s[...] = hi
        tlo_s[...] = (tfull - hi.astype(jnp.float32)).astype(jnp.bfloat16)
        acc_s[...] = jnp.zeros((_N, 24), jnp.float32)

    ids_n = nidx_ref[0, 0, 0, :]
    ids_b = bidx_ref[0, 0, 0, :]
    seg = seg_ref[0, :]

    lane = jax.lax.broadcasted_iota(jnp.int32, (_EC, 2 * _N), 1)
    oh = ((ids_n[:, None] == lane) | (ids_b[:, None] == lane)
          ).astype(jnp.bfloat16)

    g = (jnp.dot(oh, thi_s[...], preferred_element_type=jnp.float32)
         + jnp.dot(oh, tlo_s[...], preferred_element_type=jnp.float32))

    z1 = _gelu(g[:, 0:16])
    msg = _gelu(jnp.dot(z1, wp_ref[:, 0:16],
                        preferred_element_type=jnp.float32)
                + cr_ref[0, 0:16][None, :])
    cfz = _gelu(jnp.dot(msg, wp_ref[:, 16:32],
                        preferred_element_type=jnp.float32)
                + tb_ref[0, 0, 16:32][None, :])
    mc = jnp.concatenate([msg, cfz], axis=1)
    abc = _gelu(jnp.dot(mc, wcat_ref[...],
                        preferred_element_type=jnp.float32)
                + tb_ref[0, 0, 48:56][None, :])
    cu = abc[:, 2:3] * (abc[:, 0:1] * g[:, 16:20]
                        + abc[:, 1:2] * g[:, 20:24])

    sub = jax.lax.broadcasted_iota(jnp.int32, (_N, _EC), 0)
    oh_s = (sub == seg).astype(jnp.bfloat16)
    scat = jnp.concatenate(
        [msg, cu, jnp.ones((_EC, 1), jnp.float32),
         jnp.zeros((_EC, 3), jnp.float32)], axis=1)
    shi = scat.astype(jnp.bfloat16)
    slo = (scat - shi.astype(jnp.float32)).astype(jnp.bfloat16)
    acc_s[...] += (jnp.dot(oh_s, shi, preferred_element_type=jnp.float32)
                   + jnp.dot(oh_s, slo, preferred_element_type=jnp.float32))

    @pl.when(c == nc - 1)
    def _fin():
        aggm = acc_s[:, 0:16]
        aggc = acc_s[:, 16:20]
        cnt = acc_s[:, 20:21]
        hxb = hx_ref[0]
        ox_ref[0] = hxb[:, 16:20] + jnp.where(
            cnt > 0.0, aggc / jnp.maximum(cnt, 1.0), 0.0)
        zi = _gelu(jnp.dot(hxb[:, 0:16], wp_ref[:, 48:64],
                           preferred_element_type=jnp.float32)
                   + jnp.dot(aggm, wp_ref[:, 64:80],
                             preferred_element_type=jnp.float32)
                   + tb_ref[0, 0, 32:48][None, :])
        oh_ref[0] = _gelu(jnp.dot(zi, wp_ref[:, 32:48],
                                  preferred_element_type=jnp.float32)
                          + cr_ref[0, 16:32][None, :])


def kernel(x, h, edges, edge_weights, time_embed, message_params,
           coord_params, inv_params, Wa, ba, Wb, bb):
    del edge_weights
    w1p, b1p = _fold_bn(message_params[0])
    w2p, b2p = _fold_bn(message_params[1])
    wc1p, bc1p = _fold_bn(coord_params[0])
    wc2p, bc2p = _fold_bn(coord_params[1])
    wi1p, bi1p = _fold_bn(inv_params[0])
    wi2p, bi2p = _fold_bn(inv_params[1])

    bias1_t = time_embed @ w1p[40:48] + b1p
    biasc1_t = time_embed @ wc1p[16:24] + bc1p
    biasi_t = time_embed @ wi1p[32:40] + bi1p
    ca = time_embed @ Wa[16:24] + ba
    cb = time_embed @ Wb[16:24] + bb
    bc2_b = jnp.broadcast_to(bc2p[None, :], (_B, 1))
    tb = jnp.concatenate(
        [bias1_t, biasc1_t, biasi_t, ca, cb, bc2_b,
         jnp.zeros((_B, 13), jnp.float32)], axis=1).reshape(_B, 1, 64)

    w20 = jnp.concatenate(
        [jnp.concatenate([w1p[0:16], w1p[32:36]], axis=0),
         jnp.concatenate([w1p[16:32], w1p[36:40]], axis=0)], axis=1)
    wp = jnp.concatenate(
        [w2p, wc1p[0:16], wi2p, wi1p[0:16], wi1p[16:32],
         jnp.zeros((16, 16), jnp.float32)], axis=1)
    z16 = jnp.zeros((16, 1), jnp.float32)
    wcat = jnp.concatenate(
        [jnp.concatenate([Wa[0:16], Wb[0:16], z16], axis=1),
         jnp.concatenate([z16, z16, wc2p], axis=1)], axis=0)
    wcat = jnp.concatenate([wcat, jnp.zeros((32, 5), jnp.float32)],
                           axis=1)
    cr = jnp.concatenate(
        [b2p, bi2p, jnp.zeros((32,), jnp.float32)])[None, :]

    hx = jnp.concatenate([h, x], axis=2)
    nidx = edges[:, :, 0].reshape(_B, _NC, 1, _EC)
    bidx = (edges[:, :, 1] + _N).reshape(_B, _NC, 1, _EC)
    seg = edges[0, :, 0].reshape(_NC, 1, _EC)

    grid = (_B, _NC)
    ox, oh = pl.pallas_call(
        _body,
        grid=grid,
        in_specs=[
            pl.BlockSpec((1, _N, 20), lambda b, c: (b, 0, 0)),
            pl.BlockSpec((1, 1, 1, _EC), lambda b, c: (b, c, 0, 0)),
            pl.BlockSpec((1, 1, 1, _EC), lambda b, c: (b, c, 0, 0)),
            pl.BlockSpec((1, 1, _EC), lambda b, c: (c, 0, 0)),
            pl.BlockSpec((1, 1, 64), lambda b, c: (b, 0, 0)),
            pl.BlockSpec((20, 32), lambda b, c: (0, 0)),
            pl.BlockSpec((16, 96), lambda b, c: (0, 0)),
            pl.BlockSpec((32, 8), lambda b, c: (0, 0)),
            pl.BlockSpec((1, 64), lambda b, c: (0, 0)),
        ],
        out_specs=[
            pl.BlockSpec((1, _N, 4), lambda b, c: (b, 0, 0)),
            pl.BlockSpec((1, _N, 16), lambda b, c: (b, 0, 0)),
        ],
        out_shape=[
            jax.ShapeDtypeStruct((_B, _N, 4), jnp.float32),
            jax.ShapeDtypeStruct((_B, _N, 16), jnp.float32),
        ],
        scratch_shapes=[
            pltpu.VMEM((2 * _N, 24), jnp.bfloat16),
            pltpu.VMEM((2 * _N, 24), jnp.bfloat16),
            pltpu.VMEM((_N, 24), jnp.float32),
        ],
        compiler_params=pltpu.CompilerParams(
            dimension_semantics=("arbitrary", "arbitrary")),
    )(hx, nidx, bidx, seg, tb, w20, wp, wcat, cr)
    return (ox, oh)
